# baseline (speedup 1.0000x reference)
import functools

import jax
import jax.numpy as jnp
from jax import lax
from jax.experimental import pallas as pl
from jax.experimental.pallas import tpu as pltpu
from jax.experimental.pallas import tpu_sc as plsc

NC = 2
NS = 16
NW = NC * NS

CHUNK = 1024
NBUF = 4


@functools.lru_cache(maxsize=None)
def _build(total, embed_dim):
    per_w = total // NW
    n = per_w // CHUNK
    assert per_w % CHUNK == 0 and n % NBUF == 0 and n >= 2 * NBUF
    max_row = total // CHUNK - 1

    mesh = plsc.VectorSubcoreMesh(core_axis_name="c", subcore_axis_name="s")

    @functools.partial(
        pl.kernel,
        mesh=mesh,
        out_type=jax.ShapeDtypeStruct((total, embed_dim), jnp.float32),
        scratch_types=[
            pltpu.VMEM((NBUF, CHUNK), jnp.int32),
            pltpu.VMEM((NBUF * CHUNK, embed_dim), jnp.float32),
            [pltpu.SemaphoreType.DMA] * NBUF,
            [pltpu.SemaphoreType.DMA] * NBUF,
            [pltpu.SemaphoreType.DMA] * NBUF,
        ],
        compiler_params=pltpu.CompilerParams(use_tc_tiling_on_sc=False),
    )
    def emb_kernel(idx_hbm, table_hbm, out_hbm, idx_v, rows_v, isems, gsems,
                   osems):
        wid = lax.axis_index("s") * NC + lax.axis_index("c")

        def idx_row(v):
            return jnp.minimum(wid * n + v, max_row)

        def out_slice(v):
            base = pl.multiple_of((wid * n + v) * CHUNK, CHUNK)
            return out_hbm.at[pl.ds(base, CHUNK)]

        def idx_fetch(v, b):
            pltpu.async_copy(idx_hbm.at[pl.ds(idx_row(v), 1)],
                             idx_v.at[pl.ds(b, 1)], isems[b])

        def gather(v, b, drain_out):
            if drain_out:
                pltpu.make_async_copy(rows_v.at[pl.ds(b * CHUNK, CHUNK)],
                                      out_slice(v), osems[b]).wait()
            pltpu.make_async_copy(idx_hbm.at[pl.ds(idx_row(v), 1)],
                                  idx_v.at[pl.ds(b, 1)], isems[b]).wait()
            pltpu.async_copy(table_hbm.at[idx_v.at[b]],
                             rows_v.at[pl.ds(b * CHUNK, CHUNK)], gsems[b])

        def complete(v, b, prefetch):
            pltpu.make_async_copy(table_hbm.at[idx_v.at[b]],
                                  rows_v.at[pl.ds(b * CHUNK, CHUNK)],
                                  gsems[b]).wait()
            pltpu.async_copy(rows_v.at[pl.ds(b * CHUNK, CHUNK)],
                             out_slice(v), osems[b])
            if prefetch:
                idx_fetch(v + NBUF, b)

        for b in range(NBUF):
            idx_fetch(b, b)
        gather(0, 0, False)
        gather(1, 1, False)
        gather(2, 2, False)
        complete(0, 0, True)
        gather(3, 3, False)
        complete(1, 1, True)

        def body(i, carry):
            for b in range(NBUF):
                v = i * NBUF + b
                gather(v, b, True)
                complete(v - 2, (b - 2) % NBUF, True)
            return carry

        lax.fori_loop(1, n // NBUF, body, 0)

        complete(n - 2, (n - 2) % NBUF, False)
        complete(n - 1, (n - 1) % NBUF, False)
        for v in range(n - NBUF, n):
            b = v % NBUF
            pltpu.make_async_copy(rows_v.at[pl.ds(b * CHUNK, CHUNK)],
                                  out_slice(v), osems[b]).wait()
        for v in range(n, n + 2):
            b = v % NBUF
            pltpu.make_async_copy(idx_hbm.at[pl.ds(idx_row(v), 1)],
                                  idx_v.at[pl.ds(b, 1)], isems[b]).wait()

    return emb_kernel


def kernel(indices, emb_weight):
    batch, hist = indices.shape
    total = batch * hist
    embed_dim = emb_weight.shape[1]
    idx2d = indices.astype(jnp.int32).reshape(total // CHUNK, CHUNK)
    out = _build(total, embed_dim)(idx2d, emb_weight)
    return out.reshape(batch, hist, embed_dim)


# speedup vs baseline: 1.0012x; 1.0012x over previous
import functools

import jax
import jax.numpy as jnp
from jax import lax
from jax.experimental import pallas as pl
from jax.experimental.pallas import tpu as pltpu
from jax.experimental.pallas import tpu_sc as plsc

NC = 2
NS = 16
NW = NC * NS

CHUNK = 512
NBUF = 8


@functools.lru_cache(maxsize=None)
def _build(total, embed_dim):
    per_w = total // NW
    n = per_w // CHUNK
    assert per_w % CHUNK == 0 and n % NBUF == 0 and n >= 2 * NBUF
    max_row = total // CHUNK - 1

    mesh = plsc.VectorSubcoreMesh(core_axis_name="c", subcore_axis_name="s")

    @functools.partial(
        pl.kernel,
        mesh=mesh,
        out_type=jax.ShapeDtypeStruct((total, embed_dim), jnp.float32),
        scratch_types=[
            pltpu.VMEM((NBUF, CHUNK), jnp.int32),
            pltpu.VMEM((NBUF * CHUNK, embed_dim), jnp.float32),
            [pltpu.SemaphoreType.DMA] * NBUF,
            [pltpu.SemaphoreType.DMA] * NBUF,
            [pltpu.SemaphoreType.DMA] * NBUF,
        ],
        compiler_params=pltpu.CompilerParams(use_tc_tiling_on_sc=False),
    )
    def emb_kernel(idx_hbm, table_hbm, out_hbm, idx_v, rows_v, isems, gsems,
                   osems):
        wid = lax.axis_index("s") * NC + lax.axis_index("c")

        def idx_row(v):
            return jnp.minimum(wid * n + v, max_row)

        def out_slice(v):
            base = pl.multiple_of((wid * n + v) * CHUNK, CHUNK)
            return out_hbm.at[pl.ds(base, CHUNK)]

        def idx_fetch(v, b):
            pltpu.async_copy(idx_hbm.at[pl.ds(idx_row(v), 1)],
                             idx_v.at[pl.ds(b, 1)], isems[b])

        def gather(v, b, drain_out):
            if drain_out:
                pltpu.make_async_copy(rows_v.at[pl.ds(b * CHUNK, CHUNK)],
                                      out_slice(v), osems[b]).wait()
            pltpu.make_async_copy(idx_hbm.at[pl.ds(idx_row(v), 1)],
                                  idx_v.at[pl.ds(b, 1)], isems[b]).wait()
            pltpu.async_copy(table_hbm.at[idx_v.at[b]],
                             rows_v.at[pl.ds(b * CHUNK, CHUNK)], gsems[b])

        def complete(v, b, prefetch):
            pltpu.make_async_copy(table_hbm.at[idx_v.at[b]],
                                  rows_v.at[pl.ds(b * CHUNK, CHUNK)],
                                  gsems[b]).wait()
            pltpu.async_copy(rows_v.at[pl.ds(b * CHUNK, CHUNK)],
                             out_slice(v), osems[b])
            if prefetch:
                idx_fetch(v + NBUF, b)

        for b in range(NBUF):
            idx_fetch(b, b)
        for v in range(NBUF):
            gather(v, v, False)
            if v >= 2:
                complete(v - 2, v - 2, True)

        def body(i, carry):
            for b in range(NBUF):
                v = i * NBUF + b
                gather(v, b, True)
                complete(v - 2, (b - 2) % NBUF, True)
            return carry

        lax.fori_loop(1, n // NBUF, body, 0)

        complete(n - 2, (n - 2) % NBUF, False)
        complete(n - 1, (n - 1) % NBUF, False)
        for v in range(n - NBUF, n):
            b = v % NBUF
            pltpu.make_async_copy(rows_v.at[pl.ds(b * CHUNK, CHUNK)],
                                  out_slice(v), osems[b]).wait()
        for v in range(n, n + NBUF - 2):
            b = v % NBUF
            pltpu.make_async_copy(idx_hbm.at[pl.ds(idx_row(v), 1)],
                                  idx_v.at[pl.ds(b, 1)], isems[b]).wait()

    return emb_kernel


def kernel(indices, emb_weight):
    batch, hist = indices.shape
    total = batch * hist
    embed_dim = emb_weight.shape[1]
    idx2d = indices.astype(jnp.int32).reshape(total // CHUNK, CHUNK)
    out = _build(total, embed_dim)(idx2d, emb_weight)
    return out.reshape(batch, hist, embed_dim)
